# baseline (speedup 1.0000x reference)
import functools

import jax
import jax.numpy as jnp
from jax import lax
from jax.experimental import pallas as pl
from jax.experimental.pallas import tpu as pltpu
from jax.experimental.pallas import tpu_sc as plsc

_VOCAB = 1000000
_EMB = 16
_BATCH = 16384
_HIST = 200

_B = _BATCH * _HIST
_NW = 32
_IDXW = 512
_ROWS_PER_BLOCK = 4
_BLK = _ROWS_PER_BLOCK * _IDXW
_N_IDX_ROWS = _B // _IDXW
_ROWS_PER_W = _N_IDX_ROWS // _NW
_BLOCKS_PER_W = _ROWS_PER_W // _ROWS_PER_BLOCK


def _emb_kernel(idx_hbm, table_hbm, out_hbm, idx_v, rows_v, gsem, ssems):
    nc = 2
    wid = lax.axis_index("s") * nc + lax.axis_index("c")
    row_base = wid * _ROWS_PER_W

    def process(b, slot, wait_store):
        row0 = row_base + b * _ROWS_PER_BLOCK
        if wait_store:
            pltpu.make_async_copy(
                rows_v.at[slot], out_hbm.at[pl.ds(0, _BLK)], ssems.at[slot]
            ).wait()
        pltpu.sync_copy(idx_hbm.at[pl.ds(row0, _ROWS_PER_BLOCK)], idx_v.at[slot])
        copies = []
        for j in range(_ROWS_PER_BLOCK):
            copies.append(
                pltpu.async_copy(
                    table_hbm.at[idx_v.at[slot].at[j]],
                    rows_v.at[slot].at[pl.ds(j * _IDXW, _IDXW)],
                    gsem,
                )
            )
        for c in copies:
            c.wait()
        pltpu.async_copy(
            rows_v.at[slot], out_hbm.at[pl.ds(row0 * _IDXW, _BLK)], ssems.at[slot]
        )

    for s in range(2):
        process(s, s, False)

    def outer(g, carry):
        for s in range(2):
            process(2 * g + s, s, True)
        return carry

    lax.fori_loop(1, _BLOCKS_PER_W // 2, outer, 0)

    for s in range(2):
        pltpu.make_async_copy(
            rows_v.at[s], out_hbm.at[pl.ds(0, _BLK)], ssems.at[s]
        ).wait()


@jax.jit
def kernel(mask, weights):
    idx = mask.reshape(_N_IDX_ROWS, _IDXW).astype(jnp.int32)
    mesh = plsc.VectorSubcoreMesh(core_axis_name="c", subcore_axis_name="s")
    k = functools.partial(
        pl.kernel,
        mesh=mesh,
        out_type=jax.ShapeDtypeStruct((_B, _EMB), jnp.float32),
        scratch_types=[
            pltpu.VMEM((2, _ROWS_PER_BLOCK, _IDXW), jnp.int32),
            pltpu.VMEM((2, _BLK, _EMB), jnp.float32),
            pltpu.SemaphoreType.DMA,
            pltpu.SemaphoreType.DMA((2,)),
        ],
        compiler_params=pltpu.CompilerParams(use_tc_tiling_on_sc=False),
    )(_emb_kernel)
    out = k(idx, weights)
    return out.reshape(_BATCH, _HIST, _EMB)


# speedup vs baseline: 1.6901x; 1.6901x over previous
import functools

import jax
import jax.numpy as jnp
from jax import lax
from jax.experimental import pallas as pl
from jax.experimental.pallas import tpu as pltpu
from jax.experimental.pallas import tpu_sc as plsc

_VOCAB = 1000000
_EMB = 16
_BATCH = 16384
_HIST = 200

_NW = 32
_TB = _BATCH // 128
_SEGS = 8
_TPS = _TB // _SEGS
_UNITS = _HIST * _SEGS
_UPW = _UNITS // _NW


def _emb_kernel(midx, table, out5, idx_v, rows_v, tbuf, gsem, ssems):
    nc = 2
    wid = lax.axis_index("s") * nc + lax.axis_index("c")
    iota = lax.iota(jnp.int32, 16)

    def unit(i, slot, wait_store):
        u = wid * _UPW + i
        h = u // _SEGS
        seg = u % _SEGS
        if wait_store:
            for te in range(2):
                pltpu.make_async_copy(
                    tbuf.at[slot, te], out5.at[0, te, pl.ds(0, _TPS)],
                    ssems.at[slot],
                ).wait()
        pltpu.sync_copy(midx.at[h, pl.ds(seg * _TPS, _TPS)], idx_v)
        copies = []
        for t in range(_TPS):
            copies.append(
                pltpu.async_copy(
                    table.at[idx_v.at[t]],
                    rows_v.at[pl.ds(t * 128, 128)],
                    gsem,
                )
            )
        for c in copies:
            c.wait()

        def trans_t(t, carry):
            def trans_e(e, carry2):
                te = e // 8
                ei = e % 8
                base = t * 128
                for bg in range(8):
                    v = plsc.load_gather(
                        rows_v, [base + bg * 16 + iota, e + iota * 0]
                    )
                    tbuf[slot, te, t, ei, pl.ds(bg * 16, 16)] = v
                return carry2
            return lax.fori_loop(0, 16, trans_e, carry)

        lax.fori_loop(0, _TPS, trans_t, 0)

        for te in range(2):
            pltpu.async_copy(
                tbuf.at[slot, te],
                out5.at[h, te, pl.ds(seg * _TPS, _TPS)],
                ssems.at[slot],
            )

    for s in range(2):
        unit(s, s, False)

    def outer(g, carry):
        for s in range(2):
            unit(2 * g + s, s, True)
        return carry

    lax.fori_loop(1, _UPW // 2, outer, 0)

    for s in range(2):
        for te in range(2):
            pltpu.make_async_copy(
                tbuf.at[s, te], out5.at[0, te, pl.ds(0, _TPS)], ssems.at[s]
            ).wait()


@jax.jit
def kernel(mask, weights):
    midx = mask.astype(jnp.int32).T.reshape(_HIST, _TB, 128)
    mesh = plsc.VectorSubcoreMesh(core_axis_name="c", subcore_axis_name="s")
    k = functools.partial(
        pl.kernel,
        mesh=mesh,
        out_type=jax.ShapeDtypeStruct((_HIST, 2, _TB, 8, 128), jnp.float32),
        scratch_types=[
            pltpu.VMEM((_TPS, 128), jnp.int32),
            pltpu.VMEM((_TPS * 128, _EMB), jnp.float32),
            pltpu.VMEM((2, 2, _TPS, 8, 128), jnp.float32),
            pltpu.SemaphoreType.DMA,
            pltpu.SemaphoreType.DMA((2,)),
        ],
        compiler_params=pltpu.CompilerParams(
            use_tc_tiling_on_sc=False, needs_layout_passes=False
        ),
    )(_emb_kernel)
    out5 = k(midx, weights)
    return out5.transpose(2, 4, 0, 1, 3).reshape(_BATCH, _HIST, _EMB)


# speedup vs baseline: 2.0620x; 1.2200x over previous
import functools

import jax
import jax.numpy as jnp
from jax import lax
from jax.experimental import pallas as pl
from jax.experimental.pallas import tpu as pltpu
from jax.experimental.pallas import tpu_sc as plsc

_VOCAB = 1000000
_EMB = 16
_BATCH = 16384
_HIST = 200

_NW = 32
_TB = _BATCH // 128
_SEGS = 8
_TPS = _TB // _SEGS
_UNITS = _HIST * _SEGS
_UPW = _UNITS // _NW
_TE_W = _TPS * 8 * 128


def _emb_kernel(midx, table, out5, idx_v, rows_v, tbuf, gsem, ssems):
    nc = 2
    wid = lax.axis_index("s") * nc + lax.axis_index("c")
    iota = lax.iota(jnp.int32, 16)
    cols = [(d + iota) % _EMB for d in range(_EMB)]
    soffs = [(c // 8) * _TE_W + (c % 8) * 128 for c in cols]

    def unit(i, slot, wait_store):
        u = wid * _UPW + i
        h = u // _SEGS
        seg = u % _SEGS
        if wait_store:
            for te in range(2):
                pltpu.make_async_copy(
                    tbuf.at[pl.ds((2 * slot + te) * _TE_W, _TE_W)],
                    out5.at[0, te, pl.ds(0, _TE_W)],
                    ssems.at[slot],
                ).wait()
        pltpu.sync_copy(midx.at[h, pl.ds(seg * _TPS, _TPS)], idx_v)
        copies = []
        for t in range(_TPS):
            copies.append(
                pltpu.async_copy(
                    table.at[idx_v.at[t]],
                    rows_v.at[pl.ds(t * 128, 128)],
                    gsem,
                )
            )
        for c in copies:
            c.wait()

        def trans_t(t, carry):
            k0 = slot * 2 * _TE_W + t * 1024 + iota
            for bg in range(8):
                row = t * 128 + bg * 16 + iota
                k = k0 + bg * 16
                for d in range(_EMB):
                    v = plsc.load_gather(rows_v, [row, cols[d]])
                    plsc.store_scatter(tbuf, [k + soffs[d]], v)
            return carry

        lax.fori_loop(0, _TPS, trans_t, 0)

        for te in range(2):
            pltpu.async_copy(
                tbuf.at[pl.ds((2 * slot + te) * _TE_W, _TE_W)],
                out5.at[h, te, pl.ds(seg * _TE_W, _TE_W)],
                ssems.at[slot],
            )

    for s in range(2):
        unit(s, s, False)

    def outer(g, carry):
        for s in range(2):
            unit(2 * g + s, s, True)
        return carry

    lax.fori_loop(1, _UPW // 2, outer, 0)

    for s in range(2):
        for te in range(2):
            pltpu.make_async_copy(
                tbuf.at[pl.ds((2 * s + te) * _TE_W, _TE_W)],
                out5.at[0, te, pl.ds(0, _TE_W)],
                ssems.at[s],
            ).wait()


@jax.jit
def kernel(mask, weights):
    midx = mask.astype(jnp.int32).T.reshape(_HIST, _TB, 128)
    mesh = plsc.VectorSubcoreMesh(core_axis_name="c", subcore_axis_name="s")
    k = functools.partial(
        pl.kernel,
        mesh=mesh,
        out_type=jax.ShapeDtypeStruct((_HIST, 2, _SEGS * _TE_W), jnp.float32),
        scratch_types=[
            pltpu.VMEM((_TPS, 128), jnp.int32),
            pltpu.VMEM((_TPS * 128, _EMB), jnp.float32),
            pltpu.VMEM((4 * _TE_W,), jnp.float32),
            pltpu.SemaphoreType.DMA,
            pltpu.SemaphoreType.DMA((2,)),
        ],
        compiler_params=pltpu.CompilerParams(
            use_tc_tiling_on_sc=False, needs_layout_passes=False
        ),
    )(_emb_kernel)
    out5 = k(midx, weights)
    return (
        out5.reshape(_HIST, 2, _TB, 8, 128)
        .transpose(2, 4, 0, 1, 3)
        .reshape(_BATCH, _HIST, _EMB)
    )


# speedup vs baseline: 3.5422x; 1.7178x over previous
import functools

import jax
import jax.numpy as jnp
from jax import lax
from jax.experimental import pallas as pl
from jax.experimental.pallas import tpu as pltpu
from jax.experimental.pallas import tpu_sc as plsc

_VOCAB = 1000000
_EMB = 16
_BATCH = 16384
_HIST = 200

_NW = 32
_TB = _BATCH // 128
_TPS = 8
_SEGS = _TB // _TPS
_UNITS = _HIST * _SEGS
_UPW = _UNITS // _NW
_TE_W = _TPS * 8 * 128


def _emb_kernel(midx, table, out5, idx_v, rows_v, tbuf, gsems, ssems, isem):
    nc = 2
    wid = lax.axis_index("s") * nc + lax.axis_index("c")
    iota = lax.iota(jnp.int32, 16)
    cols = [(d + iota) % _EMB for d in range(_EMB)]
    soffs = [(c // 8) * _TE_W + (c % 8) * 128 for c in cols]
    u_base = wid * _UPW

    def idx_fetch(u, slot):
        h = u // _SEGS
        seg = u % _SEGS
        pltpu.async_copy(
            midx.at[h, pl.ds(seg * _TPS, _TPS)], idx_v.at[slot], isem
        )

    def idx_wait(slot):
        pltpu.make_async_copy(
            midx.at[0, pl.ds(0, _TPS)], idx_v.at[slot], isem
        ).wait()

    def fire_gathers(slot):
        for t in range(_TPS):
            pltpu.async_copy(
                table.at[idx_v.at[slot, t]],
                rows_v.at[pl.ds(slot * 1024 + t * 128, 128)],
                gsems.at[slot],
            )

    def drain_gathers(slot):
        for t in range(_TPS):
            pltpu.make_async_copy(
                table.at[idx_v.at[slot, t]],
                rows_v.at[pl.ds(slot * 1024 + t * 128, 128)],
                gsems.at[slot],
            ).wait()

    def unit(i, slot, wait_store):
        u = u_base + i
        h = u // _SEGS
        seg = u % _SEGS
        nslot = 1 - slot
        if wait_store:
            for te in range(2):
                pltpu.make_async_copy(
                    tbuf.at[pl.ds((2 * slot + te) * _TE_W, _TE_W)],
                    out5.at[0, te, pl.ds(0, _TE_W)],
                    ssems.at[slot],
                ).wait()
        drain_gathers(slot)
        idx_fetch((u + 2) % _UNITS, slot)
        idx_wait(nslot)
        fire_gathers(nslot)

        def trans_t(t):
            k0 = slot * 2 * _TE_W + t * 1024 + iota
            for bg in range(8):
                row = slot * 1024 + t * 128 + bg * 16 + iota
                k = k0 + bg * 16
                vs = [plsc.load_gather(rows_v, [row, cols[d]])
                      for d in range(_EMB)]
                for d in range(_EMB):
                    plsc.store_scatter(tbuf, [k + soffs[d]], vs[d])

        plsc.parallel_loop(0, _TPS, unroll=2)(trans_t)

        for te in range(2):
            pltpu.async_copy(
                tbuf.at[pl.ds((2 * slot + te) * _TE_W, _TE_W)],
                out5.at[h, te, pl.ds(seg * _TE_W, _TE_W)],
                ssems.at[slot],
            )

    pltpu.sync_copy(midx.at[u_base // _SEGS,
                            pl.ds((u_base % _SEGS) * _TPS, _TPS)],
                    idx_v.at[0])
    fire_gathers(0)
    idx_fetch((u_base + 1) % _UNITS, 1)

    for s in range(2):
        unit(s, s, False)

    def outer(g, carry):
        for s in range(2):
            unit(2 * g + s, s, True)
        return carry

    lax.fori_loop(1, _UPW // 2, outer, 0)

    drain_gathers(0)
    idx_wait(1)
    for s in range(2):
        for te in range(2):
            pltpu.make_async_copy(
                tbuf.at[pl.ds((2 * s + te) * _TE_W, _TE_W)],
                out5.at[0, te, pl.ds(0, _TE_W)],
                ssems.at[s],
            ).wait()


@jax.jit
def kernel(mask, weights):
    midx = mask.astype(jnp.int32).T.reshape(_HIST, _TB, 128)
    mesh = plsc.VectorSubcoreMesh(core_axis_name="c", subcore_axis_name="s")
    k = functools.partial(
        pl.kernel,
        mesh=mesh,
        out_type=jax.ShapeDtypeStruct((_HIST, 2, _SEGS * _TE_W), jnp.float32),
        scratch_types=[
            pltpu.VMEM((2, _TPS, 128), jnp.int32),
            pltpu.VMEM((2 * _TPS * 128, _EMB), jnp.float32),
            pltpu.VMEM((4 * _TE_W,), jnp.float32),
            pltpu.SemaphoreType.DMA((2,)),
            pltpu.SemaphoreType.DMA((2,)),
            pltpu.SemaphoreType.DMA,
        ],
        compiler_params=pltpu.CompilerParams(
            use_tc_tiling_on_sc=False, needs_layout_passes=False
        ),
    )(_emb_kernel)
    out5 = k(midx, weights)
    return (
        out5.reshape(_HIST, 2, _TB, 8, 128)
        .transpose(2, 4, 0, 1, 3)
        .reshape(_BATCH, _HIST, _EMB)
    )
